# baseline (speedup 1.0000x reference)
import functools

import jax
import jax.numpy as jnp
from jax import lax
from jax.experimental import pallas as pl
from jax.experimental.pallas import tpu as pltpu
from jax.experimental.pallas import tpu_sc as plsc

BATCH = 16384
FACTORS = 64
NUM_WORKERS = 32
ROWS_PER_W = BATCH // NUM_WORKERS
CHUNK = 128
NCHUNK = ROWS_PER_W // CHUNK
GROUPS_PER_CHUNK = CHUNK // 16
PAIR_ROWS = 1000000 // 2
PAIR_F = 2 * FACTORS

_mesh = plsc.VectorSubcoreMesh(core_axis_name="c", subcore_axis_name="s")


@functools.partial(
    pl.kernel,
    mesh=_mesh,
    out_type=jax.ShapeDtypeStruct((BATCH,), jnp.float32),
    scratch_types=[
        pltpu.VMEM((NCHUNK, CHUNK), jnp.int32),
        pltpu.VMEM((NCHUNK, CHUNK), jnp.int32),
        pltpu.VMEM((NCHUNK, CHUNK), jnp.int32),
        pltpu.VMEM((NCHUNK, CHUNK), jnp.int32),
        pltpu.VMEM((2, CHUNK, PAIR_F), jnp.float32),
        pltpu.VMEM((2, CHUNK, PAIR_F), jnp.float32),
        pltpu.VMEM((FACTORS,), jnp.float32),
        pltpu.VMEM((16,), jnp.float32),
        pltpu.VMEM((ROWS_PER_W,), jnp.float32),
        pltpu.SemaphoreType.DMA,
        pltpu.SemaphoreType.DMA,
    ],
    compiler_params=pltpu.CompilerParams(needs_layout_passes=False),
)
def _mf_sc(uidx_hbm, iidx_hbm, ut_hbm, it_hbm, w_hbm, b_hbm, out_hbm,
           uix_v, iix_v, upx_v, ipx_v, ru_v, ri_v, w_v, b_v, out_v,
           sem0, sem1):
    sems = (sem0, sem1)
    wid = lax.axis_index("s") * 2 + lax.axis_index("c")

    pltpu.sync_copy(uidx_hbm.at[wid], uix_v)
    pltpu.sync_copy(iidx_hbm.at[wid], iix_v)
    pltpu.sync_copy(w_hbm, w_v)
    pltpu.sync_copy(b_hbm, b_v)

    for c in range(NCHUNK):
        for k in range(CHUNK // 16):
            sl = pl.ds(k * 16, 16)
            upx_v[c, sl] = lax.shift_right_logical(uix_v[c, sl], 1)
            ipx_v[c, sl] = lax.shift_right_logical(iix_v[c, sl], 1)

    def start_chunk(c):
        buf = c % 2
        return (pltpu.async_copy(ut_hbm.at[upx_v.at[c]], ru_v.at[buf],
                                 sems[buf]),
                pltpu.async_copy(it_hbm.at[ipx_v.at[c]], ri_v.at[buf],
                                 sems[buf]))

    iota16 = lax.iota(jnp.int32, 16)
    last_lane = iota16 == 15
    b_onehot = jnp.where(iota16 == 0, b_v[...], 0.0)
    wv = [w_v[pl.ds(k * 16, 16)] for k in range(FACTORS // 16)]
    KB = FACTORS // 16

    inflight = {0: start_chunk(0)}
    for c in range(NCHUNK):
        if c + 1 < NCHUNK:
            inflight[c + 1] = start_chunk(c + 1)
        for cp in inflight.pop(c):
            cp.wait()
        buf = c % 2
        ru_c = ru_v.at[buf]
        ri_c = ri_v.at[buf]

        def group_body(g, carry, c=c, ru_c=ru_c, ri_c=ri_c):
            upar = uix_v[c, pl.ds(g * 16, 16)] & 1
            ipar = iix_v[c, pl.ds(g * 16, 16)] & 1
            for s in range(16):
                r = g * 16 + s
                lane = jnp.full((16,), s, jnp.int32)
                us = upar.at[lane].get(mode="promise_in_bounds") == 1
                its = ipar.at[lane].get(mode="promise_in_bounds") == 1
                acc = b_onehot
                for k in range(KB):
                    lo = pl.ds(k * 16, 16)
                    hi = pl.ds(FACTORS + k * 16, 16)
                    u = jnp.where(us, ru_c[r, hi], ru_c[r, lo])
                    v = jnp.where(its, ri_c[r, hi], ri_c[r, lo])
                    acc = acc + u * v * wv[k]
                tot = plsc.cumsum(acc)
                plsc.store_scatter(
                    out_v, [jnp.full((16,), c * CHUNK + 0, jnp.int32) + r],
                    tot, mask=last_lane)
            return carry

        lax.fori_loop(0, GROUPS_PER_CHUNK, group_body, 0)

    pltpu.sync_copy(out_v, out_hbm.at[pl.ds(wid * ROWS_PER_W, ROWS_PER_W)])


def kernel(user_idx, item_idx, user_table, item_table, W, b):
    uidx = user_idx.reshape(NUM_WORKERS, NCHUNK, CHUNK)
    iidx = item_idx.reshape(NUM_WORKERS, NCHUNK, CHUNK)
    ut2 = user_table.reshape(PAIR_ROWS, PAIR_F)
    it2 = item_table.reshape(PAIR_ROWS, PAIR_F)
    w = W.reshape(FACTORS)
    bvec = jnp.broadcast_to(b, (16,)).astype(jnp.float32)
    return _mf_sc(uidx, iidx, ut2, it2, w, bvec)
